# baseline (speedup 1.0000x reference)
import jax
import jax.numpy as jnp
from jax.experimental import pallas as pl
from jax.experimental.pallas import tpu as pltpu

_ROWS_PER_BLOCK = 1024
_SUB_ROWS = 32


def _cumsum_kern(x_ref, o_ref, carry_ref):
    i = pl.program_id(0)

    @pl.when(i == 0)
    def _zero_carry():
        carry_ref[...] = jnp.zeros_like(carry_ref)

    s = _SUB_ROWS
    tri = (
        jax.lax.broadcasted_iota(jnp.int32, (s, s), 0)
        >= jax.lax.broadcasted_iota(jnp.int32, (s, s), 1)
    ).astype(jnp.float32)
    carry = carry_ref[...]
    for b in range(_ROWS_PER_BLOCK // s):
        sub = x_ref[b * s : (b + 1) * s, :]
        local = jnp.dot(tri, sub, preferred_element_type=jnp.float32)
        o_ref[b * s : (b + 1) * s, :] = local + carry
        carry = carry + local[s - 1 : s, :]
    carry_ref[...] = carry


def kernel(x):
    n, d = x.shape
    r = _ROWS_PER_BLOCK
    return pl.pallas_call(
        _cumsum_kern,
        grid=(n // r,),
        in_specs=[pl.BlockSpec((r, d), lambda i: (i, 0))],
        out_specs=pl.BlockSpec((r, d), lambda i: (i, 0)),
        out_shape=jax.ShapeDtypeStruct((n, d), x.dtype),
        scratch_shapes=[pltpu.VMEM((1, d), jnp.float32)],
        compiler_params=pltpu.CompilerParams(
            dimension_semantics=("arbitrary",),
        ),
    )(x)


# speedup vs baseline: 1.0024x; 1.0024x over previous
import jax
import jax.numpy as jnp
from jax.experimental import pallas as pl
from jax.experimental.pallas import tpu as pltpu

_ROWS_PER_BLOCK = 1024
_SUB_ROWS = 64


def _cumsum_kern(x_ref, o_ref, carry_ref):
    i = pl.program_id(0)

    @pl.when(i == 0)
    def _zero_carry():
        carry_ref[...] = jnp.zeros_like(carry_ref)

    s = _SUB_ROWS
    tri = (
        jax.lax.broadcasted_iota(jnp.int32, (s, s), 0)
        >= jax.lax.broadcasted_iota(jnp.int32, (s, s), 1)
    ).astype(jnp.float32)
    carry = carry_ref[...]
    for b in range(_ROWS_PER_BLOCK // s):
        sub = x_ref[b * s : (b + 1) * s, :]
        local = jnp.dot(tri, sub, preferred_element_type=jnp.float32)
        o_ref[b * s : (b + 1) * s, :] = local + carry
        carry = carry + local[s - 1 : s, :]
    carry_ref[...] = carry


def kernel(x):
    n, d = x.shape
    r = _ROWS_PER_BLOCK
    return pl.pallas_call(
        _cumsum_kern,
        grid=(n // r,),
        in_specs=[pl.BlockSpec((r, d), lambda i: (i, 0))],
        out_specs=pl.BlockSpec((r, d), lambda i: (i, 0)),
        out_shape=jax.ShapeDtypeStruct((n, d), x.dtype),
        scratch_shapes=[pltpu.VMEM((1, d), jnp.float32)],
        compiler_params=pltpu.CompilerParams(
            dimension_semantics=("arbitrary",),
        ),
    )(x)
